# Initial kernel scaffold; baseline (speedup 1.0000x reference)
import jax
import jax.numpy as jnp
from jax.experimental import pallas as pl


def kernel(x, edge_index, eyes, W1, b1, W2, b2):
    raise NotImplementedError("write your pallas kernel here")


# speedup vs baseline: 8.1964x; 8.1964x over previous
import functools

import jax
import jax.numpy as jnp
from jax import lax
from jax.experimental import pallas as pl
from jax.experimental.pallas import tpu as pltpu
from jax.experimental.pallas import tpu_sc as plsc

N = 10000
E = 320000
D_IN = 128
D_HID = 256
D_OUT = 128
BLK = 100

NC = 2
NS = 16
NW = NC * NS
CHUNK = 128
E_PAD = ((E + NW * CHUNK - 1) // (NW * CHUNK)) * (NW * CHUNK)
DEG_CHUNKS = E_PAD // (NW * CHUNK)
AGG_CHUNKS = E_PAD // (NS * CHUNK)
RPT = 632
NP = NS * RPT

_MESH = plsc.VectorSubcoreMesh(core_axis_name="c", subcore_axis_name="s")


@functools.partial(
    pl.kernel,
    out_type=jax.ShapeDtypeStruct((2 * NP, 128), jnp.float32),
    mesh=_MESH,
    scratch_types=[
        pltpu.VMEM((CHUNK,), jnp.int32),
        pltpu.VMEM((CHUNK, 128), jnp.float32),
        pltpu.VMEM_SHARED((NP, 128), jnp.float32),
    ],
)
def _deg_call(dst_hbm, zeros_hbm, ones_hbm, out_hbm, idx_v, ones_v, acc_sh):
    cid = lax.axis_index("c")
    sid = lax.axis_index("s")
    wid = sid * NC + cid
    pltpu.sync_copy(zeros_hbm.at[pl.ds(sid * RPT, RPT)],
                    acc_sh.at[pl.ds(sid * RPT, RPT)])
    pltpu.sync_copy(ones_hbm, ones_v)
    plsc.subcore_barrier()

    def step(j, carry):
        base = (wid * DEG_CHUNKS + j) * CHUNK
        pltpu.sync_copy(dst_hbm.at[pl.ds(base, CHUNK)], idx_v)
        pltpu.sync_copy(ones_v, acc_sh.at[idx_v], add=True)
        return carry

    lax.fori_loop(0, DEG_CHUNKS, step, 0)
    plsc.subcore_barrier()
    pltpu.sync_copy(acc_sh.at[pl.ds(sid * RPT, RPT)],
                    out_hbm.at[pl.ds(cid * NP + sid * RPT, RPT)])


def _make_agg(dh):
    @functools.partial(
        pl.kernel,
        out_type=jax.ShapeDtypeStruct((2 * NP, dh), jnp.float32),
        mesh=_MESH,
        scratch_types=[
            pltpu.VMEM((CHUNK,), jnp.int32),
            pltpu.VMEM((CHUNK,), jnp.int32),
            pltpu.VMEM((CHUNK, dh), jnp.float32),
            pltpu.VMEM_SHARED((NP, dh), jnp.float32),
            pltpu.SemaphoreType.DMA,
        ],
    )
    def agg(g_hbm, src2_hbm, dst_hbm, out_hbm, sidx_v, didx_v, rows_v, acc_sh, sem):
        cid = lax.axis_index("c")
        sid = lax.axis_index("s")
        pltpu.sync_copy(g_hbm.at[pl.ds(cid * NP + sid * RPT, RPT)],
                        acc_sh.at[pl.ds(sid * RPT, RPT)])
        plsc.subcore_barrier()

        def step(j, carry):
            base = (sid * AGG_CHUNKS + j) * CHUNK
            pltpu.sync_copy(src2_hbm.at[pl.ds(cid * E_PAD + base, CHUNK)], sidx_v)
            pltpu.sync_copy(dst_hbm.at[pl.ds(base, CHUNK)], didx_v)
            pltpu.async_copy(g_hbm.at[sidx_v], rows_v, sem).wait()
            pltpu.sync_copy(rows_v, acc_sh.at[didx_v], add=True)
            return carry

        lax.fori_loop(0, AGG_CHUNKS, step, 0)
        plsc.subcore_barrier()
        pltpu.sync_copy(acc_sh.at[pl.ds(sid * RPT, RPT)],
                        out_hbm.at[pl.ds(cid * NP + sid * RPT, RPT)])

    return agg


_agg128 = _make_agg(128)

EDGE_HALF = E_PAD // NC
EDGE_CHUNKS = E_PAD // (NW * CHUNK)


@functools.partial(
    pl.kernel,
    out_type=jax.ShapeDtypeStruct((2 * NP, 128), jnp.float32),
    mesh=_MESH,
    scratch_types=[
        pltpu.VMEM((CHUNK,), jnp.int32),
        pltpu.VMEM((CHUNK,), jnp.int32),
        pltpu.VMEM((CHUNK, 128), jnp.float32),
        pltpu.VMEM_SHARED((NP, 128), jnp.float32),
        pltpu.SemaphoreType.DMA,
    ],
)
def _agg_edge(g_hbm, gh_hbm, src_hbm, dst_hbm, out_hbm,
              sidx_v, didx_v, rows_v, acc_sh, sem):
    cid = lax.axis_index("c")
    sid = lax.axis_index("s")
    pltpu.sync_copy(gh_hbm.at[pl.ds(sid * RPT, RPT)],
                    acc_sh.at[pl.ds(sid * RPT, RPT)])
    plsc.subcore_barrier()

    def step(j, carry):
        base = cid * EDGE_HALF + (sid * EDGE_CHUNKS + j) * CHUNK
        pltpu.sync_copy(src_hbm.at[pl.ds(base, CHUNK)], sidx_v)
        pltpu.sync_copy(dst_hbm.at[pl.ds(base, CHUNK)], didx_v)
        pltpu.async_copy(g_hbm.at[sidx_v], rows_v, sem).wait()
        pltpu.sync_copy(rows_v, acc_sh.at[didx_v], add=True)
        return carry

    lax.fori_loop(0, EDGE_CHUNKS, step, 0)
    plsc.subcore_barrier()
    pltpu.sync_copy(acc_sh.at[pl.ds(sid * RPT, RPT)],
                    out_hbm.at[pl.ds(cid * NP + sid * RPT, RPT)])


def _b_body(x_ref, w1_ref, b1_ref, p0_ref, p1_ref, g1_ref, dinv_ref):
    deg = 1.0 + p0_ref[0, :, :1] + p1_ref[0, :, :1]
    dinv = lax.rsqrt(deg)
    h = jnp.dot(x_ref[...], w1_ref[...], preferred_element_type=jnp.float32)
    g = dinv * (h + b1_ref[...])
    g1_ref[0] = g[:, :128]
    g1_ref[1] = g[:, 128:]
    dinv_ref[...] = dinv


def _b_call(x, w1, b1, degp):
    return pl.pallas_call(
        _b_body,
        grid=(NS,),
        in_specs=[
            pl.BlockSpec((RPT, D_IN), lambda i: (i, 0)),
            pl.BlockSpec((D_IN, D_HID), lambda i: (0, 0)),
            pl.BlockSpec((1, D_HID), lambda i: (0, 0)),
            pl.BlockSpec((1, RPT, 128), lambda i: (0, i, 0)),
            pl.BlockSpec((1, RPT, 128), lambda i: (1, i, 0)),
        ],
        out_specs=[
            pl.BlockSpec((2, RPT, 128), lambda i: (0, i, 0)),
            pl.BlockSpec((RPT, 1), lambda i: (i, 0)),
        ],
        out_shape=[
            jax.ShapeDtypeStruct((2, NP, 128), jnp.float32),
            jax.ShapeDtypeStruct((NP, 1), jnp.float32),
        ],
    )(x, w1, b1, degp, degp)


def _d_body(agg_ref, dinv_ref, w2a_ref, w2b_ref, b2_ref, g2_ref, gh_ref):
    dinv = dinv_ref[...]
    h1a = jnp.tanh(dinv * agg_ref[0])
    h1b = jnp.tanh(dinv * agg_ref[1])
    h = jnp.dot(h1a, w2a_ref[...], preferred_element_type=jnp.float32)
    h = h + jnp.dot(h1b, w2b_ref[...], preferred_element_type=jnp.float32)
    g2 = dinv * (h + b2_ref[...])
    g2_ref[...] = g2
    gh_ref[...] = 0.5 * g2


def _d_call(agg1, dinv, w2a, w2b, b2):
    return pl.pallas_call(
        _d_body,
        grid=(NS,),
        in_specs=[
            pl.BlockSpec((2, RPT, 128), lambda i: (0, i, 0)),
            pl.BlockSpec((RPT, 1), lambda i: (i, 0)),
            pl.BlockSpec((128, D_OUT), lambda i: (0, 0)),
            pl.BlockSpec((128, D_OUT), lambda i: (0, 0)),
            pl.BlockSpec((1, D_OUT), lambda i: (0, 0)),
        ],
        out_specs=[
            pl.BlockSpec((RPT, D_OUT), lambda i: (i, 0)),
            pl.BlockSpec((RPT, D_OUT), lambda i: (i, 0)),
        ],
        out_shape=[
            jax.ShapeDtypeStruct((NP, D_OUT), jnp.float32),
            jax.ShapeDtypeStruct((NP, D_OUT), jnp.float32),
        ],
    )(agg1, dinv, w2a, w2b, b2)


def _f_body(agg_ref, dinv_ref, out_ref):
    dinv = dinv_ref[0]
    h2 = jnp.tanh(dinv * (agg_ref[0, 0] + agg_ref[1, 0]))
    dn = (((1,), (1,)), ((), ()))
    p = lax.dot_general(h2, h2, dn, preferred_element_type=jnp.float32)
    row = lax.broadcasted_iota(jnp.int32, (BLK, BLK), 0)
    col = lax.broadcasted_iota(jnp.int32, (BLK, BLK), 1)
    out_ref[0] = jnp.where(row == col, 0.0, p)


def _f_call(agg2, dinv):
    return pl.pallas_call(
        _f_body,
        grid=(N // BLK,),
        in_specs=[
            pl.BlockSpec((2, 1, BLK, 128), lambda i: (0, i, 0, 0)),
            pl.BlockSpec((1, BLK, 1), lambda i: (i, 0, 0)),
        ],
        out_specs=pl.BlockSpec((1, BLK, BLK), lambda i: (i, 0, 0)),
        out_shape=jax.ShapeDtypeStruct((N // BLK, BLK, BLK), jnp.float32),
    )(agg2, dinv)


def kernel(x, edge_index, eyes, W1, b1, W2, b2):
    src = edge_index[0].astype(jnp.int32)
    dst = edge_index[1].astype(jnp.int32)
    pad = E_PAD - E
    srcp = jnp.concatenate([src, jnp.zeros((pad,), jnp.int32)])
    dstp = jnp.concatenate([dst, jnp.full((pad,), N, jnp.int32)])
    src2 = jnp.concatenate([srcp, srcp + NP])

    xp = jnp.pad(x, ((0, NP - N), (0, 0)))
    zeros_init = jnp.zeros((NP, 128), jnp.float32)
    ones_c = jnp.ones((CHUNK, 128), jnp.float32)

    degp = _deg_call(dstp, zeros_init, ones_c).reshape(2, NP, 128)
    g1, dinv = _b_call(xp, W1, b1.reshape(1, -1), degp)
    agg1 = _agg128(g1.reshape(2 * NP, 128), src2, dstp)
    g2, g2h = _d_call(agg1.reshape(2, NP, 128), dinv, W2[:128], W2[128:],
                      b2.reshape(1, -1))
    agg2 = _agg_edge(g2, g2h, srcp, dstp)
    out = _f_call(agg2.reshape(2, NP, 128)[:, :N].reshape(2, N // BLK, BLK, 128),
                  dinv[:N].reshape(N // BLK, BLK, 1))
    return out.reshape(N, BLK)
